# Initial kernel scaffold; baseline (speedup 1.0000x reference)
import jax
import jax.numpy as jnp
from jax.experimental import pallas as pl


def kernel(indices, token_table, pos_table, gamma, beta):
    raise NotImplementedError("write your pallas kernel here")


# speedup vs baseline: 2.7008x; 2.7008x over previous
import functools

import jax
import jax.numpy as jnp
from jax import lax
from jax.experimental import pallas as pl
from jax.experimental.pallas import tpu as pltpu
from jax.experimental.pallas import tpu_sc as plsc

B = 1024
S = 200
H = 128
NW = 32
SEQ_PER_W = B // NW
HALF = S // 2
LANES = 16
NCH = H // LANES

_mesh = plsc.VectorSubcoreMesh(core_axis_name="c", subcore_axis_name="s")


@functools.partial(
    pl.kernel,
    mesh=_mesh,
    out_type=jax.ShapeDtypeStruct((B, S, H), jnp.float32),
    scratch_types=[
        pltpu.VMEM((2 * SEQ_PER_W, HALF), jnp.int32),
        pltpu.VMEM((S, H), jnp.float32),
        pltpu.VMEM((S, H), jnp.float32),
        pltpu.VMEM((S, H), jnp.float32),
        pltpu.VMEM((H,), jnp.float32),
        pltpu.VMEM((H,), jnp.float32),
        pltpu.SemaphoreType.DMA,
        pltpu.SemaphoreType.DMA,
    ],
)
def _bert_embed_sc(idx_hbm, tok_hbm, pos_hbm, g_hbm, b_hbm, out_hbm,
                   idx_v, pos_v, buf0, buf1, gv, bv, gsem0, gsem1):
    w = lax.axis_index("s") * 2 + lax.axis_index("c")
    base_seq = w * SEQ_PER_W

    pltpu.sync_copy(idx_hbm.at[pl.ds(w * (2 * SEQ_PER_W), 2 * SEQ_PER_W)], idx_v)
    pltpu.sync_copy(pos_hbm.at[pl.ds(0, S)], pos_v)
    pltpu.sync_copy(g_hbm, gv)
    pltpu.sync_copy(b_hbm, bv)

    def start_gather(s_local, buf, sem):
        pltpu.async_copy(tok_hbm.at[idx_v.at[2 * s_local]],
                         buf.at[pl.ds(0, HALF)], sem)
        pltpu.async_copy(tok_hbm.at[idx_v.at[2 * s_local + 1]],
                         buf.at[pl.ds(HALF, HALF)], sem)

    def wait_gather(s_local, buf, sem):
        pltpu.make_async_copy(tok_hbm.at[idx_v.at[2 * s_local]],
                              buf.at[pl.ds(0, HALF)], sem).wait()
        pltpu.make_async_copy(tok_hbm.at[idx_v.at[2 * s_local + 1]],
                              buf.at[pl.ds(HALF, HALF)], sem).wait()

    g_regs = [gv[pl.ds(c * LANES, LANES)] for c in range(NCH)]
    b_regs = [bv[pl.ds(c * LANES, LANES)] for c in range(NCH)]

    _dn = lax.GatherDimensionNumbers(
        offset_dims=(), collapsed_slice_dims=(0,), start_index_map=(0,))
    lane = lax.iota(jnp.int32, LANES)

    def lane_sum(v):
        for shift in (8, 4, 2, 1):
            perm = (lane ^ shift).reshape(LANES, 1)
            v = v + lax.gather(v, perm, _dn, (1,),
                               mode=lax.GatherScatterMode.PROMISE_IN_BOUNDS)
        return v

    def compute(buf):
        def row(r, carry):
            e = [buf[r, pl.ds(c * LANES, LANES)] + pos_v[r, pl.ds(c * LANES, LANES)]
                 for c in range(NCH)]
            s1 = e[0]
            for c in range(1, NCH):
                s1 = s1 + e[c]
            sq = [v * v for v in e]
            s2 = sq[0]
            for c in range(1, NCH):
                s2 = s2 + sq[c]
            mean_v = lane_sum(s1) * jnp.float32(1.0 / H)
            ex2_v = lane_sum(s2) * jnp.float32(1.0 / H)
            var_v = ex2_v - mean_v * mean_v
            x = var_v + jnp.float32(1e-5)
            i = lax.bitcast_convert_type(x, jnp.int32)
            i = jnp.int32(0x5F3759DF) - (i >> 1)
            y = lax.bitcast_convert_type(i, jnp.float32)
            xh = x * jnp.float32(0.5)
            for _ in range(3):
                y = y * (jnp.float32(1.5) - xh * (y * y))
            for c in range(NCH):
                buf[r, pl.ds(c * LANES, LANES)] = (
                    (e[c] - mean_v) * (y * g_regs[c]) + b_regs[c])
            return carry
        lax.fori_loop(0, S, row, 0)

    def process(s_local, buf, sem, prefetch):
        wait_gather(s_local, buf, sem)
        compute(buf)
        pltpu.sync_copy(buf, out_hbm.at[base_seq + s_local])
        if prefetch:
            start_gather(s_local + 2, buf, sem)

    start_gather(0, buf0, gsem0)
    start_gather(1, buf1, gsem1)

    def outer(g, carry):
        process(2 * g, buf0, gsem0, True)
        process(2 * g + 1, buf1, gsem1, True)
        return carry
    lax.fori_loop(0, SEQ_PER_W // 2 - 1, outer, 0)
    process(SEQ_PER_W - 2, buf0, gsem0, False)
    process(SEQ_PER_W - 1, buf1, gsem1, False)


def kernel(indices, token_table, pos_table, gamma, beta):
    idx2 = indices.astype(jnp.int32).reshape(2 * B, HALF)
    return _bert_embed_sc(idx2, token_table, pos_table, gamma, beta)


# speedup vs baseline: 4.5084x; 1.6693x over previous
import functools

import jax
import jax.numpy as jnp
from jax import lax
from jax.experimental import pallas as pl
from jax.experimental.pallas import tpu as pltpu
from jax.experimental.pallas import tpu_sc as plsc

B = 1024
S = 200
H = 128
NW = 32
SEQ_PER_W = B // NW
HALF = S // 2
LANES = 16
NCH = H // LANES

_mesh = plsc.VectorSubcoreMesh(core_axis_name="c", subcore_axis_name="s")


@functools.partial(
    pl.kernel,
    mesh=_mesh,
    out_type=jax.ShapeDtypeStruct((B, S, H), jnp.float32),
    scratch_types=[
        pltpu.VMEM((2 * SEQ_PER_W, HALF), jnp.int32),
        pltpu.VMEM((S, H), jnp.float32),
        pltpu.VMEM((S, H), jnp.float32),
        pltpu.VMEM((S, H), jnp.float32),
        pltpu.VMEM((H,), jnp.float32),
        pltpu.VMEM((H,), jnp.float32),
        pltpu.SemaphoreType.DMA,
        pltpu.SemaphoreType.DMA,
    ],
)
def _bert_embed_sc(idx_hbm, tok_hbm, pos_hbm, g_hbm, b_hbm, out_hbm,
                   idx_v, pos_v, buf0, buf1, gv, bv, gsem0, gsem1):
    w = lax.axis_index("s") * 2 + lax.axis_index("c")
    base_seq = w * SEQ_PER_W

    pltpu.sync_copy(idx_hbm.at[pl.ds(w * (2 * SEQ_PER_W), 2 * SEQ_PER_W)], idx_v)
    pltpu.sync_copy(pos_hbm.at[pl.ds(0, S)], pos_v)
    pltpu.sync_copy(g_hbm, gv)
    pltpu.sync_copy(b_hbm, bv)

    def start_gather(s_local, buf, sem):
        pltpu.async_copy(tok_hbm.at[idx_v.at[2 * s_local]],
                         buf.at[pl.ds(0, HALF)], sem)
        pltpu.async_copy(tok_hbm.at[idx_v.at[2 * s_local + 1]],
                         buf.at[pl.ds(HALF, HALF)], sem)

    def wait_gather(s_local, buf, sem):
        pltpu.make_async_copy(tok_hbm.at[idx_v.at[2 * s_local]],
                              buf.at[pl.ds(0, HALF)], sem).wait()
        pltpu.make_async_copy(tok_hbm.at[idx_v.at[2 * s_local + 1]],
                              buf.at[pl.ds(HALF, HALF)], sem).wait()

    g_regs = [gv[pl.ds(c * LANES, LANES)] for c in range(NCH)]
    b_regs = [bv[pl.ds(c * LANES, LANES)] for c in range(NCH)]

    _dn = lax.GatherDimensionNumbers(
        offset_dims=(), collapsed_slice_dims=(0,), start_index_map=(0,))
    lane = lax.iota(jnp.int32, LANES)

    def lane_sum(v):
        for shift in (8, 4, 2, 1):
            perm = (lane ^ shift).reshape(LANES, 1)
            v = v + lax.gather(v, perm, _dn, (1,),
                               mode=lax.GatherScatterMode.PROMISE_IN_BOUNDS)
        return v

    UNROLL = 2

    def compute(buf):
        def one_row(r):
            e = [buf[r, pl.ds(c * LANES, LANES)] + pos_v[r, pl.ds(c * LANES, LANES)]
                 for c in range(NCH)]
            s1 = e[0]
            for c in range(1, NCH):
                s1 = s1 + e[c]
            sq = [v * v for v in e]
            s2 = sq[0]
            for c in range(1, NCH):
                s2 = s2 + sq[c]
            mean_v = lane_sum(s1) * jnp.float32(1.0 / H)
            ex2_v = lane_sum(s2) * jnp.float32(1.0 / H)
            var_v = ex2_v - mean_v * mean_v
            x = var_v + jnp.float32(1e-5)
            i = lax.bitcast_convert_type(x, jnp.int32)
            i = jnp.int32(0x5F3759DF) - (i >> 1)
            y = lax.bitcast_convert_type(i, jnp.float32)
            xh = x * jnp.float32(0.5)
            for _ in range(2):
                y = y * (jnp.float32(1.5) - xh * (y * y))
            for c in range(NCH):
                buf[r, pl.ds(c * LANES, LANES)] = (
                    (e[c] - mean_v) * (y * g_regs[c]) + b_regs[c])

        def row(r, carry):
            for u in range(UNROLL):
                one_row(UNROLL * r + u)
            return carry
        lax.fori_loop(0, S // UNROLL, row, 0)

    def process(s_local, buf, sem, prefetch):
        wait_gather(s_local, buf, sem)
        compute(buf)
        pltpu.sync_copy(buf, out_hbm.at[base_seq + s_local])
        if prefetch:
            start_gather(s_local + 2, buf, sem)

    start_gather(0, buf0, gsem0)
    start_gather(1, buf1, gsem1)

    def outer(g, carry):
        process(2 * g, buf0, gsem0, True)
        process(2 * g + 1, buf1, gsem1, True)
        return carry
    lax.fori_loop(0, SEQ_PER_W // 2 - 1, outer, 0)
    process(SEQ_PER_W - 2, buf0, gsem0, False)
    process(SEQ_PER_W - 1, buf1, gsem1, False)


def kernel(indices, token_table, pos_table, gamma, beta):
    idx2 = indices.astype(jnp.int32).reshape(2 * B, HALF)
    return _bert_embed_sc(idx2, token_table, pos_table, gamma, beta)


# speedup vs baseline: 4.5165x; 1.0018x over previous
import functools

import jax
import jax.numpy as jnp
from jax import lax
from jax.experimental import pallas as pl
from jax.experimental.pallas import tpu as pltpu
from jax.experimental.pallas import tpu_sc as plsc

B = 1024
S = 200
H = 128
NW = 32
SEQ_PER_W = B // NW
HALF = S // 2
LANES = 16
NCH = H // LANES

_mesh = plsc.VectorSubcoreMesh(core_axis_name="c", subcore_axis_name="s")


@functools.partial(
    pl.kernel,
    mesh=_mesh,
    out_type=jax.ShapeDtypeStruct((B, S, H), jnp.float32),
    scratch_types=[
        pltpu.VMEM((2 * SEQ_PER_W, HALF), jnp.int32),
        pltpu.VMEM((S, H), jnp.float32),
        pltpu.VMEM((S, H), jnp.float32),
        pltpu.VMEM((S, H), jnp.float32),
        pltpu.VMEM((H,), jnp.float32),
        pltpu.VMEM((H,), jnp.float32),
        pltpu.SemaphoreType.DMA,
        pltpu.SemaphoreType.DMA,
    ],
)
def _bert_embed_sc(idx_hbm, tok_hbm, pos_hbm, g_hbm, b_hbm, out_hbm,
                   idx_v, pos_v, buf0, buf1, gv, bv, gsem0, gsem1):
    w = lax.axis_index("s") * 2 + lax.axis_index("c")
    base_seq = w * SEQ_PER_W

    pltpu.sync_copy(idx_hbm.at[pl.ds(w * (2 * SEQ_PER_W), 2 * SEQ_PER_W)], idx_v)
    pltpu.sync_copy(pos_hbm.at[pl.ds(0, S)], pos_v)
    pltpu.sync_copy(g_hbm, gv)
    pltpu.sync_copy(b_hbm, bv)

    def start_gather(s_local, buf, sem):
        pltpu.async_copy(tok_hbm.at[idx_v.at[2 * s_local]],
                         buf.at[pl.ds(0, HALF)], sem)
        pltpu.async_copy(tok_hbm.at[idx_v.at[2 * s_local + 1]],
                         buf.at[pl.ds(HALF, HALF)], sem)

    def wait_gather(s_local, buf, sem):
        pltpu.make_async_copy(tok_hbm.at[idx_v.at[2 * s_local]],
                              buf.at[pl.ds(0, HALF)], sem).wait()
        pltpu.make_async_copy(tok_hbm.at[idx_v.at[2 * s_local + 1]],
                              buf.at[pl.ds(HALF, HALF)], sem).wait()

    g_regs = [gv[pl.ds(c * LANES, LANES)] for c in range(NCH)]
    b_regs = [bv[pl.ds(c * LANES, LANES)] for c in range(NCH)]

    _dn = lax.GatherDimensionNumbers(
        offset_dims=(), collapsed_slice_dims=(0,), start_index_map=(0,))
    lane = lax.iota(jnp.int32, LANES)

    def lane_sum(v):
        for shift in (8, 4, 2, 1):
            perm = (lane ^ shift).reshape(LANES, 1)
            v = v + lax.gather(v, perm, _dn, (1,),
                               mode=lax.GatherScatterMode.PROMISE_IN_BOUNDS)
        return v

    UNROLL = 4

    def compute(buf):
        def one_row(r):
            e = [buf[r, pl.ds(c * LANES, LANES)] + pos_v[r, pl.ds(c * LANES, LANES)]
                 for c in range(NCH)]
            s1 = e[0]
            for c in range(1, NCH):
                s1 = s1 + e[c]
            sq = [v * v for v in e]
            s2 = sq[0]
            for c in range(1, NCH):
                s2 = s2 + sq[c]
            mean_v = lane_sum(s1) * jnp.float32(1.0 / H)
            ex2_v = lane_sum(s2) * jnp.float32(1.0 / H)
            var_v = ex2_v - mean_v * mean_v
            x = var_v + jnp.float32(1e-5)
            i = lax.bitcast_convert_type(x, jnp.int32)
            i = jnp.int32(0x5F3759DF) - (i >> 1)
            y = lax.bitcast_convert_type(i, jnp.float32)
            xh = x * jnp.float32(0.5)
            for _ in range(2):
                y = y * (jnp.float32(1.5) - xh * (y * y))
            for c in range(NCH):
                buf[r, pl.ds(c * LANES, LANES)] = (
                    (e[c] - mean_v) * (y * g_regs[c]) + b_regs[c])

        def row(r, carry):
            for u in range(UNROLL):
                one_row(UNROLL * r + u)
            return carry
        lax.fori_loop(0, S // UNROLL, row, 0)

    def process(s_local, buf, sem, prefetch):
        wait_gather(s_local, buf, sem)
        compute(buf)
        pltpu.sync_copy(buf, out_hbm.at[base_seq + s_local])
        if prefetch:
            start_gather(s_local + 2, buf, sem)

    start_gather(0, buf0, gsem0)
    start_gather(1, buf1, gsem1)

    def outer(g, carry):
        process(2 * g, buf0, gsem0, True)
        process(2 * g + 1, buf1, gsem1, True)
        return carry
    lax.fori_loop(0, SEQ_PER_W // 2 - 1, outer, 0)
    process(SEQ_PER_W - 2, buf0, gsem0, False)
    process(SEQ_PER_W - 1, buf1, gsem1, False)


def kernel(indices, token_table, pos_table, gamma, beta):
    idx2 = indices.astype(jnp.int32).reshape(2 * B, HALF)
    return _bert_embed_sc(idx2, token_table, pos_table, gamma, beta)
